# baseline (speedup 1.0000x reference)
import functools

import jax
import jax.numpy as jnp
from jax import lax
from jax.experimental import pallas as pl
from jax.experimental.pallas import tpu as pltpu
from jax.experimental.pallas import tpu_sc as plsc

N = 10000
D_IN = 128
H = 64
E = 640000

NC = 2
NS = 16
NW = NC * NS
CW = 128
NBUF = 4
CHUNKS = 160
EP = NW * CHUNKS * CW
NPAD = 10112
RPT = NPAD // NS
PAD_ROWS = NPAD - N
DW = 16

def _deg_body(dst_hbm, ones_hbm, zeros_hbm, out_hbm, dst_v, ones_v, acc):
    c = lax.axis_index("c")
    s = lax.axis_index("s")
    w = s * NC + c
    row0 = s * RPT
    pltpu.sync_copy(dst_hbm.at[w], dst_v)
    pltpu.sync_copy(ones_hbm, ones_v)
    pltpu.sync_copy(zeros_hbm.at[pl.ds(row0, RPT)], acc.at[pl.ds(row0, RPT)])
    plsc.subcore_barrier()

    def chunk(j, carry):
        pltpu.sync_copy(ones_v, acc.at[dst_v.at[j]], add=True)
        return carry

    lax.fori_loop(0, CHUNKS, chunk, 0)
    plsc.subcore_barrier()
    pltpu.sync_copy(acc.at[pl.ds(row0, RPT)], out_hbm.at[c, pl.ds(row0, RPT)])


@functools.lru_cache(maxsize=None)
def _get_sc_deg():
    mesh = plsc.VectorSubcoreMesh(
        core_axis_name="c", subcore_axis_name="s",
        num_cores=NC, num_subcores=NS)
    return pl.kernel(
        _deg_body,
        out_type=jax.ShapeDtypeStruct((NC, NPAD, DW), jnp.float32),
        mesh=mesh,
        compiler_params=pltpu.CompilerParams(use_tc_tiling_on_sc=False),
        scratch_types=[
            pltpu.VMEM((CHUNKS, CW), jnp.int32),
            pltpu.VMEM((CW, DW), jnp.float32),
            pltpu.VMEM_SHARED((NPAD, DW), jnp.float32),
        ],
    )


def _agg_body(xs_hbm, src_hbm, dst_hbm, zeros_hbm, out_hbm,
              acc,
              src_v, dst_v,
              r0, r1, r2, r3,
              g0, g1, g2, g3):
    c = lax.axis_index("c")
    s = lax.axis_index("s")
    w = s * NC + c
    row0 = s * RPT
    pltpu.sync_copy(src_hbm.at[w], src_v)
    pltpu.sync_copy(dst_hbm.at[w], dst_v)
    pltpu.sync_copy(zeros_hbm.at[pl.ds(row0, RPT)], acc.at[pl.ds(row0, RPT)])
    plsc.subcore_barrier()

    rows = (r0, r1, r2, r3)
    gsem = (g0, g1, g2, g3)

    for b in range(3):
        pltpu.async_copy(xs_hbm.at[src_v.at[b]], rows[b], gsem[b])

    def step(jj, carry):
        base = jj * NBUF
        for b in range(NBUF):
            j = base + b
            b3 = (b + 3) % NBUF
            pltpu.make_async_copy(
                xs_hbm.at[src_v.at[j]], rows[b], gsem[b]).wait()
            pltpu.sync_copy(rows[b], acc.at[dst_v.at[j]], add=True)

            @pl.when(j + 3 < CHUNKS)
            def _():
                pltpu.async_copy(
                    xs_hbm.at[src_v.at[j + 3]], rows[b3], gsem[b3])
        return carry

    lax.fori_loop(0, CHUNKS // NBUF, step, 0)
    plsc.subcore_barrier()
    pltpu.sync_copy(acc.at[pl.ds(row0, RPT)], out_hbm.at[c, pl.ds(row0, RPT)])


@functools.lru_cache(maxsize=None)
def _get_sc_agg():
    mesh = plsc.VectorSubcoreMesh(
        core_axis_name="c", subcore_axis_name="s",
        num_cores=NC, num_subcores=NS)
    return pl.kernel(
        _agg_body,
        out_type=jax.ShapeDtypeStruct((NC, NPAD, H), jnp.float32),
        mesh=mesh,
        compiler_params=pltpu.CompilerParams(use_tc_tiling_on_sc=False),
        scratch_types=(
            [pltpu.VMEM_SHARED((NPAD, H), jnp.float32)]
            + [pltpu.VMEM((CHUNKS, CW), jnp.int32)] * 2
            + [pltpu.VMEM((CW, H), jnp.float32)] * NBUF
            + [pltpu.SemaphoreType.DMA] * NBUF
        ),
    )

R = 1000
_GRID = (N // R,)


def _dinv_of(dp):
    deg = dp[0, :, 0:1] + dp[1, :, 0:1] + 1.0
    return lax.rsqrt(deg)


def _first_body(x_ref, w_ref, dp_ref, o_ref):
    dinv = _dinv_of(dp_ref[...])
    xw = jnp.dot(x_ref[...], w_ref[...], preferred_element_type=jnp.float32)
    o_ref[...] = dinv * xw


_tc_first = pl.pallas_call(
    _first_body,
    grid=_GRID,
    in_specs=[
        pl.BlockSpec((R, D_IN), lambda i: (i, 0)),
        pl.BlockSpec((D_IN, H), lambda i: (0, 0)),
        pl.BlockSpec((NC, R, DW), lambda i: (0, i, 0)),
    ],
    out_specs=pl.BlockSpec((R, H), lambda i: (i, 0)),
    out_shape=jax.ShapeDtypeStruct((N, H), jnp.float32),
)


def _mid_body(p_ref, xs_ref, dp_ref, b_ref, w_ref, o_ref):
    dinv = _dinv_of(dp_ref[...])
    agg = p_ref[0] + p_ref[1] + xs_ref[...]
    h = jnp.maximum(dinv * agg + b_ref[...], 0.0)
    hw = jnp.dot(h, w_ref[...], preferred_element_type=jnp.float32)
    o_ref[...] = dinv * hw


_tc_mid = pl.pallas_call(
    _mid_body,
    grid=_GRID,
    in_specs=[
        pl.BlockSpec((NC, R, H), lambda i: (0, i, 0)),
        pl.BlockSpec((R, H), lambda i: (i, 0)),
        pl.BlockSpec((NC, R, DW), lambda i: (0, i, 0)),
        pl.BlockSpec((1, H), lambda i: (0, 0)),
        pl.BlockSpec((H, H), lambda i: (0, 0)),
    ],
    out_specs=pl.BlockSpec((R, H), lambda i: (i, 0)),
    out_shape=jax.ShapeDtypeStruct((N, H), jnp.float32),
)


def _pred_body(p_ref, xs_ref, dp_ref, b3_ref, wp1_ref, bp1_ref, wp2_ref,
               bp2_ref, o_ref):
    dinv = _dinv_of(dp_ref[...])
    agg = p_ref[0] + p_ref[1] + xs_ref[...]
    h3 = jnp.maximum(dinv * agg + b3_ref[...], 0.0)
    hp = jnp.maximum(
        jnp.dot(h3, wp1_ref[...], preferred_element_type=jnp.float32)
        + bp1_ref[...], 0.0)
    o_ref[...] = (
        jnp.dot(hp, wp2_ref[...], preferred_element_type=jnp.float32)
        + bp2_ref[...])


_tc_pred = pl.pallas_call(
    _pred_body,
    grid=_GRID,
    in_specs=[
        pl.BlockSpec((NC, R, H), lambda i: (0, i, 0)),
        pl.BlockSpec((R, H), lambda i: (i, 0)),
        pl.BlockSpec((NC, R, DW), lambda i: (0, i, 0)),
        pl.BlockSpec((1, H), lambda i: (0, 0)),
        pl.BlockSpec((H, H // 2), lambda i: (0, 0)),
        pl.BlockSpec((1, H // 2), lambda i: (0, 0)),
        pl.BlockSpec((H // 2, 1), lambda i: (0, 0)),
        pl.BlockSpec((1, 1), lambda i: (0, 0)),
    ],
    out_specs=pl.BlockSpec((R, 1), lambda i: (i, 0)),
    out_shape=jax.ShapeDtypeStruct((N, 1), jnp.float32),
)


def kernel(x, edge_index, batch, W1, b1, W2, b2, W3, b3, Wp1, bp1, Wp2, bp2):
    pad = EP - E
    src3 = jnp.concatenate(
        [edge_index[0], jnp.zeros((pad,), jnp.int32)]).reshape(NW, CHUNKS, CW)
    pad_dst = N + (jnp.arange(pad, dtype=jnp.int32) % PAD_ROWS)
    dst3 = jnp.concatenate([edge_index[1], pad_dst]).reshape(NW, CHUNKS, CW)
    zeros_h = jnp.zeros((NPAD, H), jnp.float32)
    zeros_d = jnp.zeros((NPAD, DW), jnp.float32)
    ones_d = jnp.ones((CW, DW), jnp.float32)

    sc_deg = _get_sc_deg()
    sc_agg = _get_sc_agg()
    degp = sc_deg(dst3, ones_d, zeros_d)
    xs1 = _tc_first(x, W1, degp)
    p1 = sc_agg(xs1, src3, dst3, zeros_h)
    xs2 = _tc_mid(p1, xs1, degp, b1.reshape(1, H), W2)
    p2 = sc_agg(xs2, src3, dst3, zeros_h)
    xs3 = _tc_mid(p2, xs2, degp, b2.reshape(1, H), W3)
    p3 = sc_agg(xs3, src3, dst3, zeros_h)
    out = _tc_pred(p3, xs3, degp, b3.reshape(1, H), Wp1,
                   bp1.reshape(1, H // 2), Wp2, bp2.reshape(1, 1))
    return out


# speedup vs baseline: 1.5546x; 1.5546x over previous
import functools

import jax
import jax.numpy as jnp
from jax import lax
from jax.experimental import pallas as pl
from jax.experimental.pallas import tpu as pltpu
from jax.experimental.pallas import tpu_sc as plsc

N = 10000
D_IN = 128
H = 64
E = 640000

NC = 2
NS = 16
NW = NC * NS
CW = 128
CHUNKS = 158
EP = NW * CHUNKS * CW
NPAD = 10112
RPT = NPAD // NS
PAD_ROW = 10008
DW = 16

def _deg_body(dst_hbm, ones_hbm, zeros_hbm, out_hbm, dst_v, ones_v, acc):
    c = lax.axis_index("c")
    s = lax.axis_index("s")
    w = s * NC + c
    row0 = s * RPT
    pltpu.sync_copy(dst_hbm.at[w], dst_v)
    pltpu.sync_copy(ones_hbm, ones_v)
    pltpu.sync_copy(zeros_hbm.at[pl.ds(row0, RPT)], acc.at[pl.ds(row0, RPT)])
    plsc.subcore_barrier()

    def chunk(j, carry):
        pltpu.sync_copy(ones_v, acc.at[dst_v.at[j]], add=True)
        return carry

    lax.fori_loop(0, CHUNKS, chunk, 0)
    plsc.subcore_barrier()
    pltpu.sync_copy(acc.at[pl.ds(row0, RPT)], out_hbm.at[c, pl.ds(row0, RPT)])


@functools.lru_cache(maxsize=None)
def _get_sc_deg():
    mesh = plsc.VectorSubcoreMesh(
        core_axis_name="c", subcore_axis_name="s",
        num_cores=NC, num_subcores=NS)
    return pl.kernel(
        _deg_body,
        out_type=jax.ShapeDtypeStruct((NC, NPAD, DW), jnp.float32),
        mesh=mesh,
        compiler_params=pltpu.CompilerParams(use_tc_tiling_on_sc=False),
        scratch_types=[
            pltpu.VMEM((CHUNKS, CW), jnp.int32),
            pltpu.VMEM((CW, DW), jnp.float32),
            pltpu.VMEM_SHARED((NPAD, DW), jnp.float32),
        ],
    )


def _agg_body(xs_hbm, src_hbm, dst_hbm, zeros_hbm, out_hbm,
              src_v, dst_v, rows0, rows1, acc, g0, g1):
    c = lax.axis_index("c")
    s = lax.axis_index("s")
    w = s * NC + c
    row0 = s * RPT
    pltpu.sync_copy(src_hbm.at[w], src_v)
    pltpu.sync_copy(dst_hbm.at[w], dst_v)
    pltpu.sync_copy(zeros_hbm.at[pl.ds(row0, RPT)], acc.at[pl.ds(row0, RPT)])
    plsc.subcore_barrier()

    pltpu.async_copy(xs_hbm.at[src_v.at[0]], rows0, g0)

    def pair(jj, carry):
        j0 = jj * 2
        pltpu.async_copy(xs_hbm.at[src_v.at[j0 + 1]], rows1, g1)
        pltpu.make_async_copy(xs_hbm.at[src_v.at[j0]], rows0, g0).wait()
        pltpu.sync_copy(rows0, acc.at[dst_v.at[j0]], add=True)

        @pl.when(j0 + 2 < CHUNKS)
        def _():
            pltpu.async_copy(xs_hbm.at[src_v.at[j0 + 2]], rows0, g0)

        pltpu.make_async_copy(xs_hbm.at[src_v.at[j0 + 1]], rows1, g1).wait()
        pltpu.sync_copy(rows1, acc.at[dst_v.at[j0 + 1]], add=True)
        return carry

    lax.fori_loop(0, CHUNKS // 2, pair, 0)
    plsc.subcore_barrier()
    pltpu.sync_copy(acc.at[pl.ds(row0, RPT)], out_hbm.at[c, pl.ds(row0, RPT)])


@functools.lru_cache(maxsize=None)
def _get_sc_agg():
    mesh = plsc.VectorSubcoreMesh(
        core_axis_name="c", subcore_axis_name="s",
        num_cores=NC, num_subcores=NS)
    return pl.kernel(
        _agg_body,
        out_type=jax.ShapeDtypeStruct((NC, NPAD, H), jnp.float32),
        mesh=mesh,
        compiler_params=pltpu.CompilerParams(use_tc_tiling_on_sc=False),
        scratch_types=[
            pltpu.VMEM((CHUNKS, CW), jnp.int32),
            pltpu.VMEM((CHUNKS, CW), jnp.int32),
            pltpu.VMEM((CW, H), jnp.float32),
            pltpu.VMEM((CW, H), jnp.float32),
            pltpu.VMEM_SHARED((NPAD, H), jnp.float32),
            pltpu.SemaphoreType.DMA,
            pltpu.SemaphoreType.DMA,
        ],
    )

R = 1000
_GRID = (N // R,)


def _dinv_of(dp):
    deg = dp[0, :, 0:1] + dp[1, :, 0:1] + 1.0
    return 1.0 / jnp.sqrt(deg)


def _first_body(x_ref, w_ref, dp_ref, o_ref):
    dinv = _dinv_of(dp_ref[...])
    xw = jnp.dot(x_ref[...], w_ref[...], preferred_element_type=jnp.float32)
    o_ref[...] = dinv * xw


_tc_first = pl.pallas_call(
    _first_body,
    grid=_GRID,
    in_specs=[
        pl.BlockSpec((R, D_IN), lambda i: (i, 0)),
        pl.BlockSpec((D_IN, H), lambda i: (0, 0)),
        pl.BlockSpec((NC, R, DW), lambda i: (0, i, 0)),
    ],
    out_specs=pl.BlockSpec((R, H), lambda i: (i, 0)),
    out_shape=jax.ShapeDtypeStruct((N, H), jnp.float32),
)


def _mid_body(p_ref, xs_ref, dp_ref, b_ref, w_ref, o_ref):
    dinv = _dinv_of(dp_ref[...])
    agg = p_ref[0] + p_ref[1] + xs_ref[...]
    h = jnp.maximum(dinv * agg + b_ref[...], 0.0)
    hw = jnp.dot(h, w_ref[...], preferred_element_type=jnp.float32)
    o_ref[...] = dinv * hw


_tc_mid = pl.pallas_call(
    _mid_body,
    grid=_GRID,
    in_specs=[
        pl.BlockSpec((NC, R, H), lambda i: (0, i, 0)),
        pl.BlockSpec((R, H), lambda i: (i, 0)),
        pl.BlockSpec((NC, R, DW), lambda i: (0, i, 0)),
        pl.BlockSpec((1, H), lambda i: (0, 0)),
        pl.BlockSpec((H, H), lambda i: (0, 0)),
    ],
    out_specs=pl.BlockSpec((R, H), lambda i: (i, 0)),
    out_shape=jax.ShapeDtypeStruct((N, H), jnp.float32),
)


def _pred_body(p_ref, xs_ref, dp_ref, b3_ref, wp1_ref, bp1_ref, wp2_ref,
               bp2_ref, o_ref):
    dinv = _dinv_of(dp_ref[...])
    agg = p_ref[0] + p_ref[1] + xs_ref[...]
    h3 = jnp.maximum(dinv * agg + b3_ref[...], 0.0)
    hp = jnp.maximum(
        jnp.dot(h3, wp1_ref[...], preferred_element_type=jnp.float32)
        + bp1_ref[...], 0.0)
    o_ref[...] = (
        jnp.dot(hp, wp2_ref[...], preferred_element_type=jnp.float32)
        + bp2_ref[...])


_tc_pred = pl.pallas_call(
    _pred_body,
    grid=_GRID,
    in_specs=[
        pl.BlockSpec((NC, R, H), lambda i: (0, i, 0)),
        pl.BlockSpec((R, H), lambda i: (i, 0)),
        pl.BlockSpec((NC, R, DW), lambda i: (0, i, 0)),
        pl.BlockSpec((1, H), lambda i: (0, 0)),
        pl.BlockSpec((H, H // 2), lambda i: (0, 0)),
        pl.BlockSpec((1, H // 2), lambda i: (0, 0)),
        pl.BlockSpec((H // 2, 1), lambda i: (0, 0)),
        pl.BlockSpec((1, 1), lambda i: (0, 0)),
    ],
    out_specs=pl.BlockSpec((R, 1), lambda i: (i, 0)),
    out_shape=jax.ShapeDtypeStruct((N, 1), jnp.float32),
)


def kernel(x, edge_index, batch, W1, b1, W2, b2, W3, b3, Wp1, bp1, Wp2, bp2):
    pad = EP - E
    src3 = jnp.concatenate(
        [edge_index[0], jnp.zeros((pad,), jnp.int32)]).reshape(NW, CHUNKS, CW)
    dst3 = jnp.concatenate(
        [edge_index[1], jnp.full((pad,), PAD_ROW, jnp.int32)]
    ).reshape(NW, CHUNKS, CW)
    zeros_h = jnp.zeros((NPAD, H), jnp.float32)
    zeros_d = jnp.zeros((NPAD, DW), jnp.float32)
    ones_d = jnp.ones((CW, DW), jnp.float32)

    sc_deg = _get_sc_deg()
    sc_agg = _get_sc_agg()
    degp = sc_deg(dst3, ones_d, zeros_d)
    xs1 = _tc_first(x, W1, degp)
    p1 = sc_agg(xs1, src3, dst3, zeros_h)
    xs2 = _tc_mid(p1, xs1, degp, b1.reshape(1, H), W2)
    p2 = sc_agg(xs2, src3, dst3, zeros_h)
    xs3 = _tc_mid(p2, xs2, degp, b2.reshape(1, H), W3)
    p3 = sc_agg(xs3, src3, dst3, zeros_h)
    out = _tc_pred(p3, xs3, degp, b3.reshape(1, H), Wp1,
                   bp1.reshape(1, H // 2), Wp2, bp2.reshape(1, 1))
    return out
